# baseline (speedup 1.0000x reference)
import jax
import jax.numpy as jnp
from jax.experimental import pallas as pl
from jax.experimental.pallas import tpu as pltpu


def _body(x_hbm, o_ref, buf, sem):
    b = pl.program_id(0)
    cp = pltpu.make_async_copy(x_hbm.at[b], buf, sem)
    cp.start()
    cp.wait()
    for p in range(9):
        seg = buf[:, 2 * p:2 * p + 2, :].reshape(1728, 160)
        t = jnp.transpose(seg, (1, 0))
        even = jnp.stack(
            [t[:80, a * 72:(a + 1) * 72] for a in range(12)], axis=1
        ).reshape(960, 72)
        odd = jnp.stack(
            [t[80:, a * 72:(a + 1) * 72] for a in range(24)], axis=1
        ).reshape(1920, 72)
        o_ref[0, p * 2880:p * 2880 + 960] = even
        o_ref[0, p * 2880 + 960:(p + 1) * 2880] = odd


def kernel(x):
    B = x.shape[0]
    out = pl.pallas_call(
        _body,
        grid=(B,),
        in_specs=[pl.BlockSpec(memory_space=pl.ANY)],
        out_specs=pl.BlockSpec((1, 25920, 72), lambda b: (b, 0, 0)),
        out_shape=jax.ShapeDtypeStruct((B, 25920, 72), jnp.float32),
        scratch_shapes=[
            pltpu.VMEM((1728, 18, 80), jnp.float32),
            pltpu.SemaphoreType.DMA,
        ],
        compiler_params=pltpu.CompilerParams(
            vmem_limit_bytes=60 * 1024 * 1024,
        ),
    )(x)
    return out


# speedup vs baseline: 2.2829x; 2.2829x over previous
import jax
import jax.numpy as jnp
from jax.experimental import pallas as pl
from jax.experimental.pallas import tpu as pltpu


def _body(x_hbm, o_ref, buf, tbuf, sem):
    b = pl.program_id(0)
    cp = pltpu.make_async_copy(x_hbm.at[b], buf, sem)
    cp.start()
    cp.wait()
    for a in range(24):
        tbuf[a] = jnp.transpose(buf[a * 72:(a + 1) * 72, :], (1, 0))
    for p in range(9):
        base = p * 160
        even = jnp.stack(
            [tbuf[a, base:base + 80, :] for a in range(12)], axis=1
        ).reshape(960, 72)
        odd = jnp.stack(
            [tbuf[a, base + 80:base + 160, :] for a in range(24)], axis=1
        ).reshape(1920, 72)
        o_ref[0, p * 2880:p * 2880 + 960] = even
        o_ref[0, p * 2880 + 960:(p + 1) * 2880] = odd


def kernel(x):
    B = x.shape[0]
    x3 = x.reshape(B, 1728, 1440)
    out = pl.pallas_call(
        _body,
        grid=(B,),
        in_specs=[pl.BlockSpec(memory_space=pl.ANY)],
        out_specs=pl.BlockSpec((1, 25920, 72), lambda b: (b, 0, 0)),
        out_shape=jax.ShapeDtypeStruct((B, 25920, 72), jnp.float32),
        scratch_shapes=[
            pltpu.VMEM((1728, 1440), jnp.float32),
            pltpu.VMEM((24, 1440, 72), jnp.float32),
            pltpu.SemaphoreType.DMA,
        ],
        compiler_params=pltpu.CompilerParams(
            vmem_limit_bytes=62 * 1024 * 1024,
        ),
    )(x3)
    return out


# speedup vs baseline: 2.5203x; 1.1040x over previous
import jax
import jax.numpy as jnp
from jax import lax
from jax.experimental import pallas as pl
from jax.experimental.pallas import tpu as pltpu


def _body(x_hbm, out_hbm, buf2, tbuf, stage2, insem, outsem):
    b = pl.program_id(0)
    nb = pl.num_programs(0)
    slot = lax.rem(b, 2)
    nslot = lax.rem(b + 1, 2)

    @pl.when(b == 0)
    def _():
        pltpu.make_async_copy(x_hbm.at[0], buf2.at[0], insem.at[0]).start()

    pltpu.make_async_copy(x_hbm.at[b], buf2.at[slot], insem.at[slot]).wait()

    @pl.when(b + 1 < nb)
    def _():
        pltpu.make_async_copy(
            x_hbm.at[b + 1], buf2.at[nslot], insem.at[nslot]
        ).start()

    for a in range(24):
        tbuf[a] = jnp.transpose(buf2[slot, a * 72:(a + 1) * 72, :], (1, 0))

    for p in range(9):
        ss = p % 2
        step = b * 9 + p

        @pl.when(step >= 2)
        def _():
            pltpu.make_async_copy(
                stage2.at[ss], out_hbm.at[b, pl.ds(0, 2880), :], outsem.at[ss]
            ).wait()

        base = p * 160
        even = jnp.stack(
            [tbuf[a, base:base + 80, :] for a in range(12)], axis=1
        ).reshape(960, 72)
        odd = jnp.stack(
            [tbuf[a, base + 80:base + 160, :] for a in range(24)], axis=1
        ).reshape(1920, 72)
        stage2[ss, :960] = even
        stage2[ss, 960:] = odd
        pltpu.make_async_copy(
            stage2.at[ss],
            out_hbm.at[b, pl.ds(p * 2880, 2880), :],
            outsem.at[ss],
        ).start()

    @pl.when(b == nb - 1)
    def _():
        for ss in (0, 1):
            pltpu.make_async_copy(
                stage2.at[ss], out_hbm.at[b, pl.ds(0, 2880), :], outsem.at[ss]
            ).wait()


def kernel(x):
    B = x.shape[0]
    x3 = x.reshape(B, 1728, 1440)
    out = pl.pallas_call(
        _body,
        grid=(B,),
        in_specs=[pl.BlockSpec(memory_space=pl.ANY)],
        out_specs=pl.BlockSpec(memory_space=pl.ANY),
        out_shape=jax.ShapeDtypeStruct((B, 25920, 72), jnp.float32),
        scratch_shapes=[
            pltpu.VMEM((2, 1728, 1440), jnp.float32),
            pltpu.VMEM((24, 1440, 72), jnp.float32),
            pltpu.VMEM((2, 2880, 72), jnp.float32),
            pltpu.SemaphoreType.DMA((2,)),
            pltpu.SemaphoreType.DMA((2,)),
        ],
        compiler_params=pltpu.CompilerParams(
            vmem_limit_bytes=62 * 1024 * 1024,
        ),
    )(x3)
    return out


# speedup vs baseline: 2.9868x; 1.1851x over previous
import jax
import jax.numpy as jnp
from jax import lax
from jax.experimental import pallas as pl
from jax.experimental.pallas import tpu as pltpu


def _body(x_hbm, out_hbm, buf2, tbuf, stage2, insem, outsem):
    b = pl.program_id(0)
    nb = pl.num_programs(0)
    slot = lax.rem(b, 2)
    nslot = lax.rem(b + 1, 2)

    @pl.when(b == 0)
    def _():
        pltpu.make_async_copy(x_hbm.at[0], buf2.at[0], insem.at[0]).start()

    pltpu.make_async_copy(x_hbm.at[b], buf2.at[slot], insem.at[slot]).wait()

    @pl.when(b + 1 < nb)
    def _():
        pltpu.make_async_copy(
            x_hbm.at[b + 1], buf2.at[nslot], insem.at[nslot]
        ).start()

    tbuf[...] = jnp.transpose(
        buf2[slot].reshape(24, 72, 1440), (2, 0, 1)
    )

    for p in range(9):
        ss = p % 2
        step = b * 9 + p

        @pl.when(step >= 2)
        def _():
            pltpu.make_async_copy(
                stage2.at[ss], out_hbm.at[b, pl.ds(0, 2880), :], outsem.at[ss]
            ).wait()

        base = p * 160
        even = tbuf[base:base + 80, :12, :].reshape(960, 72)
        odd = tbuf[base + 80:base + 160, :, :].reshape(1920, 72)
        stage2[ss, :960] = even
        stage2[ss, 960:] = odd
        pltpu.make_async_copy(
            stage2.at[ss],
            out_hbm.at[b, pl.ds(p * 2880, 2880), :],
            outsem.at[ss],
        ).start()

    @pl.when(b == nb - 1)
    def _():
        for ss in (0, 1):
            pltpu.make_async_copy(
                stage2.at[ss], out_hbm.at[b, pl.ds(0, 2880), :], outsem.at[ss]
            ).wait()


def kernel(x):
    B = x.shape[0]
    x3 = x.reshape(B, 1728, 1440)
    out = pl.pallas_call(
        _body,
        grid=(B,),
        in_specs=[pl.BlockSpec(memory_space=pl.ANY)],
        out_specs=pl.BlockSpec(memory_space=pl.ANY),
        out_shape=jax.ShapeDtypeStruct((B, 25920, 72), jnp.float32),
        scratch_shapes=[
            pltpu.VMEM((2, 1728, 1440), jnp.float32),
            pltpu.VMEM((1440, 24, 72), jnp.float32),
            pltpu.VMEM((2, 2880, 72), jnp.float32),
            pltpu.SemaphoreType.DMA((2,)),
            pltpu.SemaphoreType.DMA((2,)),
        ],
        compiler_params=pltpu.CompilerParams(
            vmem_limit_bytes=62 * 1024 * 1024,
        ),
    )(x3)
    return out


# speedup vs baseline: 3.1608x; 1.0582x over previous
import jax
import jax.numpy as jnp
from jax import lax
from jax.experimental import pallas as pl
from jax.experimental.pallas import tpu as pltpu


def _body(x_hbm, out_hbm, buf2, tbuf, stage2, insem, outsem):
    b = pl.program_id(0)
    nb = pl.num_programs(0)
    slot = lax.rem(b, 2)
    nslot = lax.rem(b + 1, 2)

    @pl.when(b == 0)
    def _():
        pltpu.make_async_copy(x_hbm.at[0], buf2.at[0], insem.at[0]).start()

    pltpu.make_async_copy(x_hbm.at[b], buf2.at[slot], insem.at[slot]).wait()

    @pl.when(b + 1 < nb)
    def _():
        pltpu.make_async_copy(
            x_hbm.at[b + 1], buf2.at[nslot], insem.at[nslot]
        ).start()

    tbuf[...] = jnp.transpose(
        buf2[slot].reshape(24, 72, 1440), (2, 0, 1)
    )

    for p in range(9):
        step = b * 9 + p
        ss = lax.rem(step, 4)

        @pl.when(step >= 4)
        def _():
            pltpu.make_async_copy(
                stage2.at[ss], out_hbm.at[b, pl.ds(0, 2880), :], outsem.at[ss]
            ).wait()

        base = p * 160
        even = tbuf[base:base + 80, :12, :].reshape(960, 72)
        odd = tbuf[base + 80:base + 160, :, :].reshape(1920, 72)
        stage2[ss, :960] = even
        stage2[ss, 960:] = odd
        pltpu.make_async_copy(
            stage2.at[ss],
            out_hbm.at[b, pl.ds(p * 2880, 2880), :],
            outsem.at[ss],
        ).start()

    @pl.when(b == nb - 1)
    def _():
        for k in range(4):
            pltpu.make_async_copy(
                stage2.at[k], out_hbm.at[b, pl.ds(0, 2880), :], outsem.at[k]
            ).wait()


def kernel(x):
    B = x.shape[0]
    x3 = x.reshape(B, 1728, 1440)
    out = pl.pallas_call(
        _body,
        grid=(B,),
        in_specs=[pl.BlockSpec(memory_space=pl.ANY)],
        out_specs=pl.BlockSpec(memory_space=pl.ANY),
        out_shape=jax.ShapeDtypeStruct((B, 25920, 72), jnp.float32),
        scratch_shapes=[
            pltpu.VMEM((2, 1728, 1440), jnp.float32),
            pltpu.VMEM((1440, 24, 72), jnp.float32),
            pltpu.VMEM((4, 2880, 72), jnp.float32),
            pltpu.SemaphoreType.DMA((2,)),
            pltpu.SemaphoreType.DMA((4,)),
        ],
        compiler_params=pltpu.CompilerParams(
            vmem_limit_bytes=62 * 1024 * 1024,
        ),
    )(x3)
    return out
